# baseline (speedup 1.0000x reference)
import functools
import math

import jax
import jax.numpy as jnp
from jax.experimental import pallas as pl
from jax.experimental.pallas import tpu as pltpu

B, N, C = 1, 2048, 768
H = 12
HD = C // H
BLKQ = 64
BLKK = 64
NQ = N // BLKQ
NK = N // BLKK
KSEL = max(1, int(0.25 * NK))
SCALE = 1.0 / math.sqrt(HD)

ROW_TILE = 256
N_ROW_TILES = N // ROW_TILE


def _qkv_kernel(x_ref, w_ref, b_ref, out_ref):
    out_ref[:, :] = jax.lax.dot_general(
        x_ref[:, :], w_ref[:, :], (((1,), (1,)), ((), ())),
        preferred_element_type=jnp.float32) + b_ref[:, :]


def _topk_kernel(p_ref, q_ref, k_ref, idx_ref):
    qb2 = jax.lax.dot_general(p_ref[:, :], q_ref[:, :], (((1,), (0,)), ((), ())),
                              preferred_element_type=jnp.float32,
                              precision=jax.lax.Precision.HIGHEST)
    kb2 = jax.lax.dot_general(p_ref[:, :], k_ref[:, :], (((1,), (0,)), ((), ())),
                              preferred_element_type=jnp.float32,
                              precision=jax.lax.Precision.HIGHEST)
    col = jax.lax.broadcasted_iota(jnp.int32, (NQ, NK), 1)
    for hh in range(2):
        qb = qb2[:, hh * HD:(hh + 1) * HD]
        kb = kb2[:, hh * HD:(hh + 1) * HD]
        s = jax.lax.dot_general(qb, kb, (((1,), (1,)), ((), ())),
                                preferred_element_type=jnp.float32)
        js = []
        for t in range(KSEL):
            m = jnp.max(s, axis=1, keepdims=True)
            j = jnp.min(jnp.where(s == m, col, NK), axis=1)
            js.append(j)
            s = jnp.where(col == j[:, None], -jnp.inf, s)
        idx_ref[hh, :, :] = jnp.stack(js, axis=1)


def _row_softmax(a):
    m = jnp.max(a, axis=1, keepdims=True)
    e = jnp.exp(a - m)
    return e / jnp.sum(e, axis=1, keepdims=True)


def _attn_kernel(idx_ref, q_ref, k_ref, v_ref, out_ref):
    g = pl.program_id(0)
    for hh in range(2):
        lo, hi = hh * HD, (hh + 1) * HD
        k = k_ref[:, lo:hi]
        v = v_ref[:, lo:hi]
        phi_k = _row_softmax(k)
        kv_tot = jax.lax.dot_general(phi_k, v, (((0,), (0,)), ((), ())),
                                     preferred_element_type=jnp.float32)
        s_tot = jnp.sum(phi_k, axis=0)

        def body(i, _):
            q_i = q_ref[pl.ds(i * BLKQ, BLKQ), lo:hi]
            ks = [k_ref[pl.ds(idx_ref[2 * g + hh, i, t] * BLKK, BLKK), lo:hi]
                  for t in range(KSEL)]
            vs = [v_ref[pl.ds(idx_ref[2 * g + hh, i, t] * BLKK, BLKK), lo:hi]
                  for t in range(KSEL)]
            k_sel = jnp.concatenate(ks, axis=0)
            v_sel = jnp.concatenate(vs, axis=0)
            s = jax.lax.dot_general(q_i, k_sel, (((1,), (1,)), ((), ())),
                                    preferred_element_type=jnp.float32) * SCALE
            e = jnp.exp(s)
            out_e = jax.lax.dot_general(e, v_sel, (((1,), (0,)), ((), ())),
                                        preferred_element_type=jnp.float32)
            den_e = jnp.sum(e, axis=1)
            pk_sel = _row_softmax(k_sel)
            kv_sel = jax.lax.dot_general(pk_sel, v_sel, (((0,), (0,)), ((), ())),
                                         preferred_element_type=jnp.float32)
            s_sel = jnp.sum(pk_sel, axis=0)
            phi_q = _row_softmax(q_i)
            out_l = jax.lax.dot_general(phi_q, kv_tot - kv_sel, (((1,), (0,)), ((), ())),
                                        preferred_element_type=jnp.float32)
            den_l = jnp.sum(phi_q * (s_tot - s_sel)[None, :], axis=1)
            den = den_e + den_l
            out_ref[pl.ds(i * BLKQ, BLKQ), lo:hi] = (out_e + out_l) / den[:, None]
            return 0

        jax.lax.fori_loop(0, NQ, body, 0)


def _ln_proj_kernel(x_ref, g_ref, bt_ref, w_ref, b_ref, out_ref):
    xb = x_ref[:, :]
    mu = jnp.mean(xb, axis=1, keepdims=True)
    d = xb - mu
    var = jnp.mean(d * d, axis=1, keepdims=True)
    xn = d / jnp.sqrt(var + 1e-5) * g_ref[:, :] + bt_ref[:, :]
    out_ref[:, :] = jax.lax.dot_general(
        xn, w_ref[:, :], (((1,), (1,)), ((), ())),
        preferred_element_type=jnp.float32) + b_ref[:, :]


@functools.partial(jax.jit, static_argnames=("interpret",))
def _run(x, W_qkv, q_bias, v_bias, gamma, beta, W_proj, b_proj, interpret=False):
    x2d = x.reshape(N, C)
    qkv_bias = jnp.concatenate([q_bias, jnp.zeros_like(q_bias), v_bias]).reshape(1, 3 * C)

    qkv = pl.pallas_call(
        _qkv_kernel,
        grid=(N_ROW_TILES,),
        in_specs=[
            pl.BlockSpec((ROW_TILE, C), lambda r: (r, 0)),
            pl.BlockSpec((3 * C, C), lambda r: (0, 0)),
            pl.BlockSpec((1, 3 * C), lambda r: (0, 0)),
        ],
        out_specs=pl.BlockSpec((ROW_TILE, 3 * C), lambda r: (r, 0)),
        out_shape=jax.ShapeDtypeStruct((N, 3 * C), jnp.float32),
        interpret=interpret,
    )(x2d, W_qkv, qkv_bias)

    pool = (jnp.kron(jnp.eye(NQ, dtype=jnp.float32),
                     jnp.ones((1, BLKQ), jnp.float32)) / BLKQ)

    idx = pl.pallas_call(
        _topk_kernel,
        grid=(H // 2,),
        in_specs=[
            pl.BlockSpec((NQ, N), lambda g: (0, 0)),
            pl.BlockSpec((N, 2 * HD), lambda g: (0, g)),
            pl.BlockSpec((N, 2 * HD), lambda g: (0, H // 2 + g)),
        ],
        out_specs=pl.BlockSpec((2, NQ, KSEL), lambda g: (g, 0, 0)),
        out_shape=jax.ShapeDtypeStruct((H, NQ, KSEL), jnp.int32),
        interpret=interpret,
    )(pool, qkv, qkv)

    x2 = pl.pallas_call(
        _attn_kernel,
        grid_spec=pltpu.PrefetchScalarGridSpec(
            num_scalar_prefetch=1,
            grid=(H // 2,),
            in_specs=[
                pl.BlockSpec((N, 2 * HD), lambda g, s: (0, g)),
                pl.BlockSpec((N, 2 * HD), lambda g, s: (0, H // 2 + g)),
                pl.BlockSpec((N, 2 * HD), lambda g, s: (0, H + g)),
            ],
            out_specs=pl.BlockSpec((N, 2 * HD), lambda g, s: (0, g)),
        ),
        out_shape=jax.ShapeDtypeStruct((N, C), jnp.float32),
        interpret=interpret,
    )(idx, qkv, qkv, qkv)

    y = pl.pallas_call(
        _ln_proj_kernel,
        grid=(N_ROW_TILES,),
        in_specs=[
            pl.BlockSpec((ROW_TILE, C), lambda r: (r, 0)),
            pl.BlockSpec((1, C), lambda r: (0, 0)),
            pl.BlockSpec((1, C), lambda r: (0, 0)),
            pl.BlockSpec((C, C), lambda r: (0, 0)),
            pl.BlockSpec((1, C), lambda r: (0, 0)),
        ],
        out_specs=pl.BlockSpec((ROW_TILE, C), lambda r: (r, 0)),
        out_shape=jax.ShapeDtypeStruct((N, C), jnp.float32),
        interpret=interpret,
    )(x2, gamma.reshape(1, C), beta.reshape(1, C), W_proj, b_proj.reshape(1, C))

    return y.reshape(B, N, C)


def kernel(x, W_qkv, q_bias, v_bias, gamma, beta, W_proj, b_proj):
    return _run(x, W_qkv, q_bias, v_bias, gamma, beta, W_proj, b_proj)


# speedup vs baseline: 1.9294x; 1.9294x over previous
import functools
import math

import jax
import jax.numpy as jnp
from jax.experimental import pallas as pl
from jax.experimental.pallas import tpu as pltpu

B, N, C = 1, 2048, 768
H = 12
HD = C // H
BLKQ = 64
BLKK = 64
NQ = N // BLKQ
NK = N // BLKK
KSEL = max(1, int(0.25 * NK))
SCALE = 1.0 / math.sqrt(HD)

ROW_TILE = 256
N_ROW_TILES = N // ROW_TILE


def _qkv_kernel(x_ref, w_ref, b_ref, out_ref):
    out_ref[:, :] = jax.lax.dot_general(
        x_ref[:, :], w_ref[:, :], (((1,), (1,)), ((), ())),
        preferred_element_type=jnp.float32) + b_ref[:, :]


def _topk_kernel(p_ref, q_ref, k_ref, idx_ref):
    qb2 = jax.lax.dot_general(p_ref[:, :], q_ref[:, :], (((1,), (0,)), ((), ())),
                              preferred_element_type=jnp.float32,
                              precision=jax.lax.Precision.HIGHEST)
    kb2 = jax.lax.dot_general(p_ref[:, :], k_ref[:, :], (((1,), (0,)), ((), ())),
                              preferred_element_type=jnp.float32,
                              precision=jax.lax.Precision.HIGHEST)
    col = jax.lax.broadcasted_iota(jnp.int32, (NQ, NK), 1)
    for hh in range(2):
        qb = qb2[:, hh * HD:(hh + 1) * HD]
        kb = kb2[:, hh * HD:(hh + 1) * HD]
        s = jax.lax.dot_general(qb, kb, (((1,), (1,)), ((), ())),
                                preferred_element_type=jnp.float32)
        js = []
        for t in range(KSEL):
            m = jnp.max(s, axis=1, keepdims=True)
            j = jnp.min(jnp.where(s == m, col, NK), axis=1)
            js.append(j)
            s = jnp.where(col == j[:, None], -jnp.inf, s)
        idx_ref[hh, :, :] = jnp.stack(js, axis=1)


def _row_softmax(a):
    m = jnp.max(a, axis=1, keepdims=True)
    e = jnp.exp(a - m)
    return e / jnp.sum(e, axis=1, keepdims=True)


def _attn_kernel(idx_ref, q_ref, k_ref, v_ref, out_ref, phi_ref, kvb_ref, sb_ref):
    g = pl.program_id(0)
    kv_tots = []
    s_tots = []
    for hh in range(2):
        lo, hi = hh * HD, (hh + 1) * HD
        k = k_ref[:, lo:hi]
        v = v_ref[:, lo:hi]
        phi = _row_softmax(k)
        phi_ref[:, lo:hi] = phi
        kv_tots.append(jax.lax.dot_general(phi, v, (((0,), (0,)), ((), ())),
                                           preferred_element_type=jnp.float32))
        s_tots.append(jnp.sum(phi, axis=0)[None, :])
        for j in range(NK):
            sl = pl.ds(j * BLKK, BLKK)
            phi_j = phi[j * BLKK:(j + 1) * BLKK, :]
            v_j = v[j * BLKK:(j + 1) * BLKK, :]
            kvb_ref[sl, lo:hi] = jax.lax.dot_general(
                phi_j, v_j, (((0,), (0,)), ((), ())),
                preferred_element_type=jnp.float32)
            sb_ref[j:j + 1, lo:hi] = jnp.sum(phi_j, axis=0)[None, :]

    def body(i, _):
        for hh in range(2):
            lo, hi = hh * HD, (hh + 1) * HD
            q_i = q_ref[pl.ds(i * BLKQ, BLKQ), lo:hi]
            idxs = [idx_ref[2 * g + hh, i, t] for t in range(KSEL)]
            k_sel = jnp.concatenate(
                [k_ref[pl.ds(j * BLKK, BLKK), lo:hi] for j in idxs], axis=0)
            v_sel = jnp.concatenate(
                [v_ref[pl.ds(j * BLKK, BLKK), lo:hi] for j in idxs], axis=0)
            s = jax.lax.dot_general(q_i, k_sel, (((1,), (1,)), ((), ())),
                                    preferred_element_type=jnp.float32) * SCALE
            e = jnp.exp(s)
            out_e = jax.lax.dot_general(e, v_sel, (((1,), (0,)), ((), ())),
                                        preferred_element_type=jnp.float32)
            den_e = jnp.sum(e, axis=1)
            kv_sel = kvb_ref[pl.ds(idxs[0] * BLKK, BLKK), lo:hi]
            s_sel = sb_ref[pl.ds(idxs[0], 1), lo:hi]
            for j in idxs[1:]:
                kv_sel = kv_sel + kvb_ref[pl.ds(j * BLKK, BLKK), lo:hi]
                s_sel = s_sel + sb_ref[pl.ds(j, 1), lo:hi]
            phi_q = _row_softmax(q_i)
            out_l = jax.lax.dot_general(phi_q, kv_tots[hh] - kv_sel,
                                        (((1,), (0,)), ((), ())),
                                        preferred_element_type=jnp.float32)
            den_l = jnp.sum(phi_q * (s_tots[hh] - s_sel), axis=1)
            den = den_e + den_l
            out_ref[pl.ds(i * BLKQ, BLKQ), lo:hi] = (out_e + out_l) / den[:, None]
        return 0

    jax.lax.fori_loop(0, NQ, body, 0)


def _ln_proj_kernel(x_ref, g_ref, bt_ref, w_ref, b_ref, out_ref):
    xb = x_ref[:, :]
    mu = jnp.mean(xb, axis=1, keepdims=True)
    d = xb - mu
    var = jnp.mean(d * d, axis=1, keepdims=True)
    xn = d / jnp.sqrt(var + 1e-5) * g_ref[:, :] + bt_ref[:, :]
    out_ref[:, :] = jax.lax.dot_general(
        xn, w_ref[:, :], (((1,), (1,)), ((), ())),
        preferred_element_type=jnp.float32) + b_ref[:, :]


@functools.partial(jax.jit, static_argnames=("interpret",))
def _run(x, W_qkv, q_bias, v_bias, gamma, beta, W_proj, b_proj, interpret=False):
    x2d = x.reshape(N, C)
    qkv_bias = jnp.concatenate([q_bias, jnp.zeros_like(q_bias), v_bias]).reshape(1, 3 * C)

    qkv = pl.pallas_call(
        _qkv_kernel,
        grid=(N_ROW_TILES,),
        in_specs=[
            pl.BlockSpec((ROW_TILE, C), lambda r: (r, 0)),
            pl.BlockSpec((3 * C, C), lambda r: (0, 0)),
            pl.BlockSpec((1, 3 * C), lambda r: (0, 0)),
        ],
        out_specs=pl.BlockSpec((ROW_TILE, 3 * C), lambda r: (r, 0)),
        out_shape=jax.ShapeDtypeStruct((N, 3 * C), jnp.float32),
        interpret=interpret,
    )(x2d, W_qkv, qkv_bias)

    pool = (jnp.kron(jnp.eye(NQ, dtype=jnp.float32),
                     jnp.ones((1, BLKQ), jnp.float32)) / BLKQ)

    idx = pl.pallas_call(
        _topk_kernel,
        grid=(H // 2,),
        in_specs=[
            pl.BlockSpec((NQ, N), lambda g: (0, 0)),
            pl.BlockSpec((N, 2 * HD), lambda g: (0, g)),
            pl.BlockSpec((N, 2 * HD), lambda g: (0, H // 2 + g)),
        ],
        out_specs=pl.BlockSpec((2, NQ, KSEL), lambda g: (g, 0, 0)),
        out_shape=jax.ShapeDtypeStruct((H, NQ, KSEL), jnp.int32),
        interpret=interpret,
    )(pool, qkv, qkv)

    x2 = pl.pallas_call(
        _attn_kernel,
        grid_spec=pltpu.PrefetchScalarGridSpec(
            num_scalar_prefetch=1,
            grid=(H // 2,),
            in_specs=[
                pl.BlockSpec((N, 2 * HD), lambda g, s: (0, g)),
                pl.BlockSpec((N, 2 * HD), lambda g, s: (0, H // 2 + g)),
                pl.BlockSpec((N, 2 * HD), lambda g, s: (0, H + g)),
            ],
            out_specs=pl.BlockSpec((N, 2 * HD), lambda g, s: (0, g)),
            scratch_shapes=[
                pltpu.VMEM((N, 2 * HD), jnp.float32),
                pltpu.VMEM((N, 2 * HD), jnp.float32),
                pltpu.VMEM((NK, 2 * HD), jnp.float32),
            ],
        ),
        out_shape=jax.ShapeDtypeStruct((N, C), jnp.float32),
        interpret=interpret,
    )(idx, qkv, qkv, qkv)

    y = pl.pallas_call(
        _ln_proj_kernel,
        grid=(N_ROW_TILES,),
        in_specs=[
            pl.BlockSpec((ROW_TILE, C), lambda r: (r, 0)),
            pl.BlockSpec((1, C), lambda r: (0, 0)),
            pl.BlockSpec((1, C), lambda r: (0, 0)),
            pl.BlockSpec((C, C), lambda r: (0, 0)),
            pl.BlockSpec((1, C), lambda r: (0, 0)),
        ],
        out_specs=pl.BlockSpec((ROW_TILE, C), lambda r: (r, 0)),
        out_shape=jax.ShapeDtypeStruct((N, C), jnp.float32),
        interpret=interpret,
    )(x2, gamma.reshape(1, C), beta.reshape(1, C), W_proj, b_proj.reshape(1, C))

    return y.reshape(B, N, C)


def kernel(x, W_qkv, q_bias, v_bias, gamma, beta, W_proj, b_proj):
    return _run(x, W_qkv, q_bias, v_bias, gamma, beta, W_proj, b_proj)


# speedup vs baseline: 2.1605x; 1.1198x over previous
import functools
import math

import jax
import jax.numpy as jnp
from jax.experimental import pallas as pl
from jax.experimental.pallas import tpu as pltpu

B, N, C = 1, 2048, 768
H = 12
HD = C // H
BLKQ = 64
BLKK = 64
NQ = N // BLKQ
NK = N // BLKK
KSEL = max(1, int(0.25 * NK))
SCALE = 1.0 / math.sqrt(HD)

ROW_TILE = 256
N_ROW_TILES = N // ROW_TILE
POOL_PER_TILE = ROW_TILE // BLKQ
G = H // 2
LN_ROWS = N // 2


def _row_softmax(a):
    m = jnp.max(a, axis=1, keepdims=True)
    e = jnp.exp(a - m)
    return e / jnp.sum(e, axis=1, keepdims=True)


def _qkv_topk_kernel(x_ref, w_ref, b_ref, qkv_ref, idx_ref, pacc_ref):
    r = pl.program_id(0)
    t = jax.lax.dot_general(x_ref[:, :], w_ref[:, :], (((1,), (1,)), ((), ())),
                            preferred_element_type=jnp.float32) + b_ref[:, :]
    qkv_ref[:, :] = t
    qk = t[:, :2 * C]
    for b in range(POOL_PER_TILE):
        row = (jnp.sum(qk[b * BLKQ:(b + 1) * BLKQ, :], axis=0) * (1.0 / BLKQ))
        pacc_ref[pl.ds(r * POOL_PER_TILE + b, 1), :] = row[None, :]

    @pl.when(r == N_ROW_TILES - 1)
    def _():
        col = jax.lax.broadcasted_iota(jnp.int32, (NQ, NK), 1)
        for h in range(H):
            qp = pacc_ref[:, h * HD:(h + 1) * HD]
            kp = pacc_ref[:, C + h * HD:C + (h + 1) * HD]
            s = jax.lax.dot_general(qp, kp, (((1,), (1,)), ((), ())),
                                    preferred_element_type=jnp.float32)
            js = []
            for t_ in range(KSEL):
                m = jnp.max(s, axis=1, keepdims=True)
                j = jnp.min(jnp.where(s == m, col, NK), axis=1)
                js.append(j)
                s = jnp.where(col == j[:, None], -jnp.inf, s)
            idx_ref[h, :, :] = jnp.stack(js, axis=1)


def _attn_ln_proj_kernel(idx_ref, q_ref, k_ref, v_ref, g_ref, bt_ref, wp_ref,
                         bp_ref, y_ref, x2_ref, phi_ref, kvb_ref, sb_ref):
    g = pl.program_id(0)

    @pl.when(g < G)
    def _attn():
        kv_tots = []
        s_tots = []
        for hh in range(2):
            lo, hi = hh * HD, (hh + 1) * HD
            k = k_ref[:, lo:hi]
            v = v_ref[:, lo:hi]
            phi = _row_softmax(k)
            phi_ref[:, lo:hi] = phi
            kv_tots.append(jax.lax.dot_general(phi, v, (((0,), (0,)), ((), ())),
                                               preferred_element_type=jnp.float32))
            s_tots.append(jnp.sum(phi, axis=0)[None, :])
            for j in range(NK):
                phi_j = phi[j * BLKK:(j + 1) * BLKK, :]
                v_j = v[j * BLKK:(j + 1) * BLKK, :]
                kvb_ref[pl.ds(j * BLKK, BLKK), lo:hi] = jax.lax.dot_general(
                    phi_j, v_j, (((0,), (0,)), ((), ())),
                    preferred_element_type=jnp.float32)
                sb_ref[j:j + 1, lo:hi] = jnp.sum(phi_j, axis=0)[None, :]

        def body(i, _):
            for hh in range(2):
                lo, hi = hh * HD, (hh + 1) * HD
                q_i = q_ref[pl.ds(i * BLKQ, BLKQ), lo:hi]
                idxs = [idx_ref[2 * g + hh, i, t] for t in range(KSEL)]
                k_sel = jnp.concatenate(
                    [k_ref[pl.ds(j * BLKK, BLKK), lo:hi] for j in idxs], axis=0)
                v_sel = jnp.concatenate(
                    [v_ref[pl.ds(j * BLKK, BLKK), lo:hi] for j in idxs], axis=0)
                s = jax.lax.dot_general(q_i, k_sel, (((1,), (1,)), ((), ())),
                                        preferred_element_type=jnp.float32) * SCALE
                e = jnp.exp(s)
                out_e = jax.lax.dot_general(e, v_sel, (((1,), (0,)), ((), ())),
                                            preferred_element_type=jnp.float32)
                den_e = jnp.sum(e, axis=1)
                kv_sel = kvb_ref[pl.ds(idxs[0] * BLKK, BLKK), lo:hi]
                s_sel = sb_ref[pl.ds(idxs[0], 1), lo:hi]
                for j in idxs[1:]:
                    kv_sel = kv_sel + kvb_ref[pl.ds(j * BLKK, BLKK), lo:hi]
                    s_sel = s_sel + sb_ref[pl.ds(j, 1), lo:hi]
                phi_q = _row_softmax(q_i)
                out_l = jax.lax.dot_general(phi_q, kv_tots[hh] - kv_sel,
                                            (((1,), (0,)), ((), ())),
                                            preferred_element_type=jnp.float32)
                den_l = jnp.sum(phi_q * (s_tots[hh] - s_sel), axis=1)
                den = den_e + den_l
                x2_ref[g, pl.ds(i * BLKQ, BLKQ), lo:hi] = (
                    (out_e + out_l) / den[:, None])
            return 0

        jax.lax.fori_loop(0, NQ, body, 0)

    @pl.when(g >= G)
    def _ln_proj():
        half = g - G
        xb = jnp.concatenate(
            [x2_ref[gg, pl.ds(half * LN_ROWS, LN_ROWS), :] for gg in range(G)],
            axis=1)
        mu = jnp.mean(xb, axis=1, keepdims=True)
        d = xb - mu
        var = jnp.mean(d * d, axis=1, keepdims=True)
        xn = d / jnp.sqrt(var + 1e-5) * g_ref[:, :] + bt_ref[:, :]
        y_ref[:, :] = jax.lax.dot_general(
            xn, wp_ref[:, :], (((1,), (1,)), ((), ())),
            preferred_element_type=jnp.float32) + bp_ref[:, :]


@functools.partial(jax.jit, static_argnames=("interpret",))
def _run(x, W_qkv, q_bias, v_bias, gamma, beta, W_proj, b_proj, interpret=False):
    x2d = x.reshape(N, C)
    qkv_bias = jnp.concatenate(
        [q_bias, jnp.zeros_like(q_bias), v_bias]).reshape(1, 3 * C)

    qkv, idx = pl.pallas_call(
        _qkv_topk_kernel,
        grid=(N_ROW_TILES,),
        in_specs=[
            pl.BlockSpec((ROW_TILE, C), lambda r: (r, 0)),
            pl.BlockSpec((3 * C, C), lambda r: (0, 0)),
            pl.BlockSpec((1, 3 * C), lambda r: (0, 0)),
        ],
        out_specs=[
            pl.BlockSpec((ROW_TILE, 3 * C), lambda r: (r, 0)),
            pl.BlockSpec((H, NQ, KSEL), lambda r: (0, 0, 0)),
        ],
        out_shape=[
            jax.ShapeDtypeStruct((N, 3 * C), jnp.float32),
            jax.ShapeDtypeStruct((H, NQ, KSEL), jnp.int32),
        ],
        scratch_shapes=[pltpu.VMEM((NQ, 2 * C), jnp.float32)],
        interpret=interpret,
    )(x2d, W_qkv, qkv_bias)

    y = pl.pallas_call(
        _attn_ln_proj_kernel,
        grid_spec=pltpu.PrefetchScalarGridSpec(
            num_scalar_prefetch=1,
            grid=(G + 2,),
            in_specs=[
                pl.BlockSpec((N, 2 * HD), lambda g, s: (0, jnp.minimum(g, G - 1))),
                pl.BlockSpec((N, 2 * HD), lambda g, s: (0, G + jnp.minimum(g, G - 1))),
                pl.BlockSpec((N, 2 * HD), lambda g, s: (0, 2 * G + jnp.minimum(g, G - 1))),
                pl.BlockSpec((1, C), lambda g, s: (0, 0)),
                pl.BlockSpec((1, C), lambda g, s: (0, 0)),
                pl.BlockSpec((C, C), lambda g, s: (0, 0)),
                pl.BlockSpec((1, C), lambda g, s: (0, 0)),
            ],
            out_specs=pl.BlockSpec((LN_ROWS, C),
                                   lambda g, s: (jnp.maximum(g - G, 0), 0)),
            scratch_shapes=[
                pltpu.VMEM((G, N, 2 * HD), jnp.float32),
                pltpu.VMEM((N, 2 * HD), jnp.float32),
                pltpu.VMEM((N, 2 * HD), jnp.float32),
                pltpu.VMEM((NK, 2 * HD), jnp.float32),
            ],
        ),
        out_shape=jax.ShapeDtypeStruct((N, C), jnp.float32),
        interpret=interpret,
    )(idx, qkv, qkv, qkv, gamma.reshape(1, C), beta.reshape(1, C),
      W_proj, b_proj.reshape(1, C))

    return y.reshape(B, N, C)


def kernel(x, W_qkv, q_bias, v_bias, gamma, beta, W_proj, b_proj):
    return _run(x, W_qkv, q_bias, v_bias, gamma, beta, W_proj, b_proj)
